# Initial kernel scaffold; baseline (speedup 1.0000x reference)
import jax
import jax.numpy as jnp
from jax.experimental import pallas as pl


def kernel(node_feat, edge_feat, rbf, state_feat, edge_index, ew1, eb1, ew2, eb2, egw1, egb1, egw2, egb2, edge_rbf_w, nw1, nb1, nw2, nb2, ngw1, ngb1, ngw2, ngb2, node_rbf_w):
    raise NotImplementedError("write your pallas kernel here")


# speedup vs baseline: 2.3786x; 2.3786x over previous
import functools

import jax
import jax.numpy as jnp
from jax import lax
from jax.experimental import pallas as pl
from jax.experimental.pallas import tpu as pltpu
from jax.experimental.pallas import tpu_sc as plsc

_N = 10000
_E = 320000
_D = 128
_DEG = 9

_NC, _NS = 2, 16
_NW = _NC * _NS
_EPW = _E // _NW
_CH = 80
_NCHUNK = _EPW // _CH
_RPT = 624
_REM = _N - _NS * _RPT

@functools.cache
def _gather_pk():
    mesh = plsc.VectorSubcoreMesh(
        core_axis_name="c", subcore_axis_name="s",
        num_cores=_NC, num_subcores=_NS)

    @functools.partial(
        pl.kernel,
        out_type=(jax.ShapeDtypeStruct((_E, _D), jnp.float32),
                  jax.ShapeDtypeStruct((_E, _D), jnp.float32)),
        mesh=mesh,
        scratch_types=[
            pltpu.VMEM((_CH,), jnp.int32),
            pltpu.VMEM((_CH,), jnp.int32),
            pltpu.VMEM((_CH, _D), jnp.float32),
            pltpu.VMEM((_CH, _D), jnp.float32),
            pltpu.SemaphoreType.DMA,
            pltpu.SemaphoreType.DMA,
        ],
    )
    def gather_k(node_hbm, src_hbm, dst_hbm, vi_hbm, vj_hbm,
                 sidx, didx, arows, brows, sem_a, sem_b):
        wid = lax.axis_index("s") * _NC + lax.axis_index("c")

        def body(k, carry):
            base = wid * _EPW + k * _CH
            pltpu.sync_copy(src_hbm.at[pl.ds(base, _CH)], sidx)
            pltpu.sync_copy(dst_hbm.at[pl.ds(base, _CH)], didx)
            cp_a = pltpu.async_copy(node_hbm.at[sidx], arows, sem_a)
            cp_b = pltpu.async_copy(node_hbm.at[didx], brows, sem_b)
            cp_a.wait()
            cp_b.wait()
            pltpu.sync_copy(arows, vi_hbm.at[pl.ds(base, _CH)])
            pltpu.sync_copy(brows, vj_hbm.at[pl.ds(base, _CH)])
            return carry

        lax.fori_loop(0, _NCHUNK, body, 0)

    return gather_k


_BLK = 1280

_dot = functools.partial(
    jax.lax.dot_general,
    dimension_numbers=(((1,), (0,)), ((), ())),
    precision=jax.lax.Precision.DEFAULT,
    preferred_element_type=jnp.float32)


def _mlp_body(vi_ref, vj_ref, ef_ref, rbf_ref,
              we1_ref, eb1_ref, we2_ref, eb2_ref, erw_ref,
              wn1_ref, nb1_ref, wn2_ref, nb2_ref, nrw_ref,
              new_e_ref, mess_ref):
    vi = vi_ref[...]
    vj = vj_ref[...]
    ef = ef_ref[...]
    rbf = rbf_ref[...]

    hg = (_dot(vi, we1_ref[0:_D, :]) + _dot(vj, we1_ref[_D:2 * _D, :])
          + _dot(ef, we1_ref[2 * _D:3 * _D, :]) + eb1_ref[...])
    hg = hg * jax.nn.sigmoid(hg)
    hg2 = _dot(hg, we2_ref[...]) + eb2_ref[...]
    h2 = hg2[:, :_D]
    h2 = h2 * jax.nn.sigmoid(h2)
    g2 = jax.nn.sigmoid(hg2[:, _D:])
    new_e = ef + h2 * g2 * _dot(rbf, erw_ref[...])
    new_e_ref[...] = new_e

    hgn = (_dot(vi, wn1_ref[0:_D, :]) + _dot(vj, wn1_ref[_D:2 * _D, :])
           + _dot(new_e, wn1_ref[2 * _D:3 * _D, :]) + nb1_ref[...])
    hgn = hgn * jax.nn.sigmoid(hgn)
    hgn2 = _dot(hgn, wn2_ref[...]) + nb2_ref[...]
    h2n = hgn2[:, :_D]
    h2n = h2n * jax.nn.sigmoid(h2n)
    g2n = jax.nn.sigmoid(hgn2[:, _D:])
    mess_ref[...] = h2n * g2n * _dot(rbf, nrw_ref[...])


def _edge_spec():
    return pl.BlockSpec((_BLK, _D), lambda i: (i, 0))


def _const_spec(shape):
    return pl.BlockSpec(shape, lambda i: tuple(0 for _ in shape))


_mlp_call = pl.pallas_call(
    _mlp_body,
    grid=(_E // _BLK,),
    in_specs=[
        _edge_spec(), _edge_spec(), _edge_spec(),
        pl.BlockSpec((_BLK, _DEG), lambda i: (i, 0)),
        _const_spec((3 * _D, 2 * _D)), _const_spec((1, 2 * _D)),
        _const_spec((2 * _D, 2 * _D)), _const_spec((1, 2 * _D)),
        _const_spec((_DEG, _D)),
        _const_spec((3 * _D, 2 * _D)), _const_spec((1, 2 * _D)),
        _const_spec((2 * _D, 2 * _D)), _const_spec((1, 2 * _D)),
        _const_spec((_DEG, _D)),
    ],
    out_specs=[_edge_spec(), _edge_spec()],
    out_shape=[jax.ShapeDtypeStruct((_E, _D), jnp.float32),
               jax.ShapeDtypeStruct((_E, _D), jnp.float32)],
)


@functools.cache
def _scatter_pk():
    mesh = plsc.VectorSubcoreMesh(
        core_axis_name="c", subcore_axis_name="s",
        num_cores=_NC, num_subcores=_NS)

    @functools.partial(
        pl.kernel,
        out_type=(jax.ShapeDtypeStruct((_N, _D), jnp.float32),
                  jax.ShapeDtypeStruct((_N, _D), jnp.float32)),
        mesh=mesh,
        scratch_types=[
            pltpu.VMEM((_CH,), jnp.int32),
            pltpu.VMEM((_CH, _D), jnp.float32),
            pltpu.VMEM_SHARED((_N, _D), jnp.float32),
        ],
    )
    def scatter_k(mess_hbm, dst_hbm, node_hbm, p0_hbm, p1_hbm, idx, rows, acc):
        c = lax.axis_index("c")
        s = lax.axis_index("s")
        wid = s * _NC + c
        row0 = s * _RPT

        pltpu.sync_copy(node_hbm.at[pl.ds(row0, _RPT)],
                        acc.at[pl.ds(row0, _RPT)])

        @pl.when(s == _NS - 1)
        def _():
            pltpu.sync_copy(node_hbm.at[pl.ds(_NS * _RPT, _REM)],
                            acc.at[pl.ds(_NS * _RPT, _REM)])

        plsc.subcore_barrier()

        def body(k, carry):
            base = wid * _EPW + k * _CH
            pltpu.sync_copy(dst_hbm.at[pl.ds(base, _CH)], idx)
            pltpu.sync_copy(mess_hbm.at[pl.ds(base, _CH)], rows)
            pltpu.sync_copy(rows, acc.at[idx], add=True)
            return carry

        lax.fori_loop(0, _NCHUNK, body, 0)
        plsc.subcore_barrier()

        @pl.when(c == 0)
        def _():
            pltpu.sync_copy(acc.at[pl.ds(row0, _RPT)],
                            p0_hbm.at[pl.ds(row0, _RPT)])

            @pl.when(s == _NS - 1)
            def _():
                pltpu.sync_copy(acc.at[pl.ds(_NS * _RPT, _REM)],
                                p0_hbm.at[pl.ds(_NS * _RPT, _REM)])

        @pl.when(c == 1)
        def _():
            pltpu.sync_copy(acc.at[pl.ds(row0, _RPT)],
                            p1_hbm.at[pl.ds(row0, _RPT)])

            @pl.when(s == _NS - 1)
            def _():
                pltpu.sync_copy(acc.at[pl.ds(_NS * _RPT, _REM)],
                                p1_hbm.at[pl.ds(_NS * _RPT, _REM)])

    return scatter_k


_CBLK = 1000


def _combine_body(p0_ref, p1_ref, nf_ref, out_ref):
    out_ref[...] = p0_ref[...] + p1_ref[...] - nf_ref[...]


_combine_call = pl.pallas_call(
    _combine_body,
    grid=(_N // _CBLK,),
    in_specs=[pl.BlockSpec((_CBLK, _D), lambda i: (i, 0))] * 3,
    out_specs=pl.BlockSpec((_CBLK, _D), lambda i: (i, 0)),
    out_shape=jax.ShapeDtypeStruct((_N, _D), jnp.float32),
)


def kernel(node_feat, edge_feat, rbf, state_feat, edge_index,
           ew1, eb1, ew2, eb2, egw1, egb1, egw2, egb2, edge_rbf_w,
           nw1, nb1, nw2, nb2, ngw1, ngb1, ngw2, ngb2, node_rbf_w):
    src = edge_index[0].astype(jnp.int32)
    dst = edge_index[1].astype(jnp.int32)

    vi, vj = _gather_pk()(node_feat, src, dst)

    zz = jnp.zeros((_D, _D), jnp.float32)
    we1 = jnp.concatenate([ew1, egw1], axis=1)
    we2 = jnp.concatenate(
        [jnp.concatenate([ew2, zz], axis=1),
         jnp.concatenate([zz, egw2], axis=1)], axis=0)
    eb1c = jnp.concatenate([eb1, egb1])[None, :]
    eb2c = jnp.concatenate([eb2, egb2])[None, :]
    wn1 = jnp.concatenate([nw1, ngw1], axis=1)
    wn2 = jnp.concatenate(
        [jnp.concatenate([nw2, zz], axis=1),
         jnp.concatenate([zz, ngw2], axis=1)], axis=0)
    nb1c = jnp.concatenate([nb1, ngb1])[None, :]
    nb2c = jnp.concatenate([nb2, ngb2])[None, :]

    new_e, mess = _mlp_call(
        vi, vj, edge_feat, rbf,
        we1, eb1c, we2, eb2c, edge_rbf_w,
        wn1, nb1c, wn2, nb2c, node_rbf_w)

    p0, p1 = _scatter_pk()(mess, dst, node_feat)
    new_v = _combine_call(p0, p1, node_feat)
    return new_e, new_v, state_feat
